# baseline (speedup 1.0000x reference)
import jax
import jax.numpy as jnp
from jax.experimental import pallas as pl
from jax.experimental.pallas import tpu as pltpu

DAMPING = 0.1


def _pos_add_kernel(x_ref, p_ref, o_ref):
    o_ref[...] = x_ref[...] + (p_ref[...] * DAMPING)[None, :, None, :]


def kernel(x, pos_table):
    B, R, C, E = x.shape
    return pl.pallas_call(
        _pos_add_kernel,
        grid=(B,),
        in_specs=[
            pl.BlockSpec((1, R, C, E), lambda b: (b, 0, 0, 0)),
            pl.BlockSpec((R, E), lambda b: (0, 0)),
        ],
        out_specs=pl.BlockSpec((1, R, C, E), lambda b: (b, 0, 0, 0)),
        out_shape=jax.ShapeDtypeStruct(x.shape, x.dtype),
        compiler_params=pltpu.CompilerParams(
            dimension_semantics=("parallel",),
        ),
    )(x, pos_table)
